# baseline (speedup 1.0000x reference)
import functools

import jax
import jax.numpy as jnp
from jax import lax
from jax.experimental import pallas as pl
from jax.experimental.pallas import tpu as pltpu
from jax.experimental.pallas import tpu_sc as plsc

_VOCAB = 1000000
_EMB = 32
_BATCH = 16384
_HIST = 50
_N = _BATCH * _HIST
_PACK = 4
_TROWS = _VOCAB // _PACK

_NC = 2
_NS = 16
_NW = _NC * _NS
_ROWS_PER_W = _BATCH // _NW
_IDX_PER_W = _ROWS_PER_W * _HIST
_RCHUNK = 4
_CHUNK = _RCHUNK * _HIST
_NCHUNK = _ROWS_PER_W // _RCHUNK
_QGROUPS = -(-_CHUNK // 16)
_HGROUPS = -(-_HIST // 16)


def _compute_q(c, idx_v, q_ref):
    base = c * _CHUNK
    lanes = lax.iota(jnp.int32, 16)

    @pl.loop(0, _QGROUPS - 1)
    def _(k):
        q_ref[pl.ds(k * 16, 16)] = lax.shift_right_logical(
            idx_v[pl.ds(base + k * 16, 16)], 2)

    tail = (_QGROUPS - 1) * 16
    vals = lax.shift_right_logical(idx_v[pl.ds(base + tail, 16)], 2)
    plsc.store_scatter(q_ref, [tail + lanes], vals,
                       mask=lanes < (_CHUNK - tail))


def _select_chunk(c, idx_v, g_ref, sel_ref):
    base = c * _CHUNK
    lanes = lax.iota(jnp.int32, 16)
    for k in range(_QGROUPS):
        r0 = k * 16
        cnt = min(16, _CHUNK - r0)
        rows16 = lanes + r0
        idx16 = idx_v[pl.ds(base + r0, 16)]
        foff16 = (idx16 & (_PACK - 1)) * _EMB
        if cnt == 16:
            mask = None
        else:
            mask = lanes < cnt
            rows16 = jnp.minimum(rows16, _CHUNK - 1)

        @pl.loop(0, _EMB, unroll=8)
        def _col(j):
            jd16 = (lanes + j) & (_EMB - 1)
            vals = plsc.load_gather(g_ref, [rows16, foff16 + jd16])
            plsc.store_scatter(sel_ref, [rows16, jd16], vals, mask=mask)


def _body(idx_hbm, tab_hbm, out_hbm,
          idx_v, q0, q1, g0, g1, sel0, sel1,
          si, sg0, sg1, ss0, ss1):
    wid = lax.axis_index("s") * _NC + lax.axis_index("c")
    ibase = wid * _IDX_PER_W
    obase = wid * _ROWS_PER_W
    q_v = (q0, q1)
    g_v = (g0, g1)
    sel_v = (sel0, sel1)
    sem_g = (sg0, sg1)
    sem_s = (ss0, ss1)

    def gather(b):
        return pltpu.async_copy(tab_hbm.at[q_v[b]], g_v[b], sem_g[b])

    def store(c, b):
        return pltpu.async_copy(
            sel_v[b].reshape(_RCHUNK, _HIST, _EMB),
            out_hbm.at[pl.ds(obase + c * _RCHUNK, _RCHUNK)],
            sem_s[b])

    def chunk_steady(c, b):
        nb = 1 - b

        @pl.when(c + 1 < _NCHUNK)
        def _():
            _compute_q(c + 1, idx_v, q_v[nb])
            gather(nb)

        pltpu.make_async_copy(tab_hbm.at[q_v[b]], g_v[b], sem_g[b]).wait()

        @pl.when(c >= 2)
        def _():
            pltpu.make_async_copy(
                sel_v[b].reshape(_RCHUNK, _HIST, _EMB),
                out_hbm.at[pl.ds(obase, _RCHUNK)], sem_s[b]).wait()

        _select_chunk(c, idx_v, g_v[b], sel_v[b])
        store(c, b)

    pltpu.async_copy(idx_hbm.at[pl.ds(ibase, _IDX_PER_W)],
                     idx_v.at[pl.ds(0, _IDX_PER_W)], si).wait()
    _compute_q(0, idx_v, q_v[0])
    gather(0)

    @pl.loop(0, _NCHUNK, step=2)
    def _(c):
        chunk_steady(c, 0)
        chunk_steady(c + 1, 1)
    pltpu.make_async_copy(
        sel_v[0].reshape(_RCHUNK, _HIST, _EMB),
        out_hbm.at[pl.ds(obase, _RCHUNK)], sem_s[0]).wait()
    pltpu.make_async_copy(
        sel_v[1].reshape(_RCHUNK, _HIST, _EMB),
        out_hbm.at[pl.ds(obase, _RCHUNK)], sem_s[1]).wait()


@functools.partial(
    pl.kernel,
    mesh=plsc.VectorSubcoreMesh(core_axis_name="c", subcore_axis_name="s"),
    compiler_params=pltpu.CompilerParams(needs_layout_passes=False),
    out_type=jax.ShapeDtypeStruct((_BATCH, _HIST, _EMB), jnp.float32),
    scratch_types=[
        pltpu.VMEM((_IDX_PER_W + 64,), jnp.int32),
        pltpu.VMEM((_CHUNK,), jnp.int32),
        pltpu.VMEM((_CHUNK,), jnp.int32),
        pltpu.VMEM((_CHUNK, 128), jnp.float32),
        pltpu.VMEM((_CHUNK, 128), jnp.float32),
        pltpu.VMEM((_CHUNK, _EMB), jnp.float32),
        pltpu.VMEM((_CHUNK, _EMB), jnp.float32),
        pltpu.SemaphoreType.DMA,
        pltpu.SemaphoreType.DMA,
        pltpu.SemaphoreType.DMA,
        pltpu.SemaphoreType.DMA,
        pltpu.SemaphoreType.DMA,
    ],
)
def _embed_lookup(idx_hbm, tab_hbm, out_hbm,
                  idx_v, q0, q1, g0, g1, sel0, sel1,
                  si, sg0, sg1, ss0, ss1):
    _body(idx_hbm, tab_hbm, out_hbm,
          idx_v, q0, q1, g0, g1, sel0, sel1,
          si, sg0, sg1, ss0, ss1)


def kernel(x, table):
    idx = x.reshape(_N).astype(jnp.int32)
    tab = table.reshape(_TROWS, _PACK * _EMB)
    return _embed_lookup(idx, tab)


# speedup vs baseline: 1.0359x; 1.0359x over previous
import functools

import jax
import jax.numpy as jnp
from jax import lax
from jax.experimental import pallas as pl
from jax.experimental.pallas import tpu as pltpu
from jax.experimental.pallas import tpu_sc as plsc

_VOCAB = 1000000
_EMB = 32
_BATCH = 16384
_HIST = 50
_N = _BATCH * _HIST
_PACK = 4
_TROWS = _VOCAB // _PACK

_NC = 2
_NS = 16
_NW = _NC * _NS
_ROWS_PER_W = _BATCH // _NW
_IDX_PER_W = _ROWS_PER_W * _HIST
_RCHUNK = 4
_CHUNK = _RCHUNK * _HIST
_NCHUNK = _ROWS_PER_W // _RCHUNK
_QGROUPS = -(-_CHUNK // 16)
_HGROUPS = -(-_HIST // 16)


def _compute_q(c, idx_v, q_ref):
    base = c * _CHUNK
    lanes = lax.iota(jnp.int32, 16)

    @pl.loop(0, _QGROUPS - 1)
    def _(k):
        q_ref[pl.ds(k * 16, 16)] = lax.shift_right_logical(
            idx_v[pl.ds(base + k * 16, 16)], 2)

    tail = (_QGROUPS - 1) * 16
    vals = lax.shift_right_logical(idx_v[pl.ds(base + tail, 16)], 2)
    plsc.store_scatter(q_ref, [tail + lanes], vals,
                       mask=lanes < (_CHUNK - tail))


def _select_chunk(c, idx_v, g_ref, sel_ref):
    base = c * _CHUNK
    lanes = lax.iota(jnp.int32, 16)
    for k in range(_QGROUPS):
        r0 = k * 16
        cnt = min(16, _CHUNK - r0)
        rows16 = lanes + r0
        idx16 = idx_v[pl.ds(base + r0, 16)]
        foff16 = (idx16 & (_PACK - 1)) * _EMB
        if cnt == 16:
            mask = None
        else:
            mask = lanes < cnt
            rows16 = jnp.minimum(rows16, _CHUNK - 1)

        @pl.loop(0, _EMB, unroll=8)
        def _col(j):
            jd16 = (lanes + j) & (_EMB - 1)
            vals = plsc.load_gather(g_ref, [rows16, foff16 + jd16])
            plsc.store_scatter(sel_ref, [rows16, jd16], vals, mask=mask)


def _body(idx_hbm, tab_hbm, out_hbm,
          idx_v, q0, q1, g0, g1, sel0, sel1,
          si, sg0, sg1, ss0, ss1):
    wid = lax.axis_index("s") * _NC + lax.axis_index("c")
    ibase = wid * _IDX_PER_W
    obase = wid * _ROWS_PER_W
    q_v = (q0, q1)
    g_v = (g0, g1)
    sel_v = (sel0, sel1)
    sem_g = (sg0, sg1)
    sem_s = (ss0, ss1)

    def gather(b):
        return pltpu.async_copy(tab_hbm.at[q_v[b]], g_v[b], sem_g[b])

    def store(c, b):
        return pltpu.async_copy(
            sel_v[b].reshape(_RCHUNK, _HIST, _EMB),
            out_hbm.at[pl.ds(obase + c * _RCHUNK, _RCHUNK)],
            sem_s[b])

    def chunk_steady(c, b):
        nb = 1 - b

        @pl.when(c + 1 < _NCHUNK)
        def _():
            _compute_q(c + 1, idx_v, q_v[nb])
            gather(nb)

        pltpu.make_async_copy(tab_hbm.at[q_v[b]], g_v[b], sem_g[b]).wait()

        @pl.when(c >= 2)
        def _():
            pltpu.make_async_copy(
                sel_v[b].reshape(_RCHUNK, _HIST, _EMB),
                out_hbm.at[pl.ds(obase, _RCHUNK)], sem_s[b]).wait()

        _select_chunk(c, idx_v, g_v[b], sel_v[b])
        store(c, b)

    pltpu.async_copy(idx_hbm.at[pl.ds(ibase, _IDX_PER_W)],
                     idx_v.at[pl.ds(0, _IDX_PER_W)], si).wait()
    _compute_q(0, idx_v, q_v[0])
    gather(0)

    @pl.loop(0, _NCHUNK, step=2)
    def _(c):
        chunk_steady(c, 0)
        chunk_steady(c + 1, 1)
    pltpu.make_async_copy(
        sel_v[0].reshape(_RCHUNK, _HIST, _EMB),
        out_hbm.at[pl.ds(obase, _RCHUNK)], sem_s[0]).wait()
    pltpu.make_async_copy(
        sel_v[1].reshape(_RCHUNK, _HIST, _EMB),
        out_hbm.at[pl.ds(obase, _RCHUNK)], sem_s[1]).wait()


@functools.partial(
    pl.kernel,
    mesh=plsc.VectorSubcoreMesh(core_axis_name="c", subcore_axis_name="s"),
    compiler_params=pltpu.CompilerParams(needs_layout_passes=False),
    out_type=jax.ShapeDtypeStruct((_BATCH, _HIST, _EMB), jnp.float32),
    scratch_types=[
        pltpu.VMEM((_IDX_PER_W + 64,), jnp.int32),
        pltpu.VMEM((_CHUNK,), jnp.int32),
        pltpu.VMEM((_CHUNK,), jnp.int32),
        pltpu.VMEM((_CHUNK, 128), jnp.float32),
        pltpu.VMEM((_CHUNK, 128), jnp.float32),
        pltpu.VMEM((_CHUNK, _EMB), jnp.float32),
        pltpu.VMEM((_CHUNK, _EMB), jnp.float32),
        pltpu.SemaphoreType.DMA,
        pltpu.SemaphoreType.DMA,
        pltpu.SemaphoreType.DMA,
        pltpu.SemaphoreType.DMA,
        pltpu.SemaphoreType.DMA,
    ],
)
def _embed_lookup(idx_hbm, tab_hbm, out_hbm,
                  idx_v, q0, q1, g0, g1, sel0, sel1,
                  si, sg0, sg1, ss0, ss1):
    _body(idx_hbm, tab_hbm, out_hbm,
          idx_v, q0, q1, g0, g1, sel0, sel1,
          si, sg0, sg1, ss0, ss1)


def kernel(x, table):
    idx = x.reshape(_N).astype(jnp.int32)
    tab = (table.T.reshape(_EMB, _TROWS, _PACK)
           .transpose(1, 2, 0).reshape(_TROWS, _PACK * _EMB))
    return _embed_lookup(idx, tab)


# speedup vs baseline: 1.3927x; 1.3444x over previous
import functools

import jax
import jax.numpy as jnp
from jax import lax
from jax.experimental import pallas as pl
from jax.experimental.pallas import tpu as pltpu
from jax.experimental.pallas import tpu_sc as plsc

_VOCAB = 1000000
_EMB = 32
_BATCH = 16384
_HIST = 50
_N = _BATCH * _HIST
_PACK = 4
_TROWS = _VOCAB // _PACK

_NC = 2
_NS = 16
_NW = _NC * _NS
_BBLK = _BATCH // _NW
_CHUNK = 256
_NCHUNK = _HIST * (_BBLK // _CHUNK)
_GROUPS = _CHUNK // 16


def _compute_q(h, bb0, idx_v, q_ref):
    @pl.loop(0, _GROUPS)
    def _(k):
        q_ref[pl.ds(k * 16, 16)] = lax.shift_right_logical(
            idx_v[h, pl.ds(bb0 + k * 16, 16)], 2)


def _select_chunk(h, bb0, idx_v, g_ref, sel_ref):
    lanes = lax.iota(jnp.int32, 16)
    for k in range(_GROUPS):
        rows16 = lanes + k * 16
        idx16 = idx_v[h, pl.ds(bb0 + k * 16, 16)]
        foff16 = (idx16 & (_PACK - 1)) * _EMB

        @pl.loop(0, _EMB, unroll=8)
        def _col(j):
            ed16 = (lanes + j) & (_EMB - 1)
            vals = plsc.load_gather(g_ref, [rows16, foff16 + ed16])
            plsc.store_scatter(sel_ref, [ed16, rows16], vals)


def _body(xt_hbm, tab_hbm, out_hbm,
          idx_v, q0, q1, g0, g1, sel0, sel1,
          si, sg0, sg1, ss0, ss1):
    wid = lax.axis_index("s") * _NC + lax.axis_index("c")
    bbase = wid * _BBLK
    q_v = (q0, q1)
    g_v = (g0, g1)
    sel_v = (sel0, sel1)
    sem_g = (sg0, sg1)
    sem_s = (ss0, ss1)

    def hb(c):
        return lax.shift_right_logical(c, 1), (c & 1) * _CHUNK

    def gather(b):
        return pltpu.async_copy(tab_hbm.at[q_v[b]], g_v[b], sem_g[b])

    def store(c, b):
        h, bb0 = hb(c)
        return pltpu.async_copy(
            sel_v[b], out_hbm.at[h, :, pl.ds(bbase + bb0, _CHUNK)],
            sem_s[b])

    def chunk_steady(c, b):
        nb = 1 - b

        @pl.when(c + 1 < _NCHUNK)
        def _():
            h1, bb1 = hb(c + 1)
            _compute_q(h1, bb1, idx_v, q_v[nb])
            gather(nb)

        pltpu.make_async_copy(tab_hbm.at[q_v[b]], g_v[b], sem_g[b]).wait()

        @pl.when(c >= 2)
        def _():
            pltpu.make_async_copy(
                sel_v[b], out_hbm.at[0, :, pl.ds(bbase, _CHUNK)],
                sem_s[b]).wait()

        h, bb0 = hb(c)
        _select_chunk(h, bb0, idx_v, g_v[b], sel_v[b])
        store(c, b)

    pltpu.async_copy(xt_hbm.at[:, pl.ds(bbase, _BBLK)], idx_v, si).wait()
    _compute_q(0, 0, idx_v, q_v[0])
    gather(0)

    @pl.loop(0, _NCHUNK, step=2)
    def _(c):
        chunk_steady(c, 0)
        chunk_steady(c + 1, 1)

    pltpu.make_async_copy(
        sel_v[0], out_hbm.at[0, :, pl.ds(bbase, _CHUNK)], sem_s[0]).wait()
    pltpu.make_async_copy(
        sel_v[1], out_hbm.at[0, :, pl.ds(bbase, _CHUNK)], sem_s[1]).wait()


@functools.partial(
    pl.kernel,
    mesh=plsc.VectorSubcoreMesh(core_axis_name="c", subcore_axis_name="s"),
    compiler_params=pltpu.CompilerParams(needs_layout_passes=False),
    out_type=jax.ShapeDtypeStruct((_HIST, _EMB, _BATCH), jnp.float32),
    scratch_types=[
        pltpu.VMEM((_HIST, _BBLK), jnp.int32),
        pltpu.VMEM((_CHUNK,), jnp.int32),
        pltpu.VMEM((_CHUNK,), jnp.int32),
        pltpu.VMEM((_CHUNK, 128), jnp.float32),
        pltpu.VMEM((_CHUNK, 128), jnp.float32),
        pltpu.VMEM((_EMB, _CHUNK), jnp.float32),
        pltpu.VMEM((_EMB, _CHUNK), jnp.float32),
        pltpu.SemaphoreType.DMA,
        pltpu.SemaphoreType.DMA,
        pltpu.SemaphoreType.DMA,
        pltpu.SemaphoreType.DMA,
        pltpu.SemaphoreType.DMA,
    ],
)
def _embed_lookup(xt_hbm, tab_hbm, out_hbm,
                  idx_v, q0, q1, g0, g1, sel0, sel1,
                  si, sg0, sg1, ss0, ss1):
    _body(xt_hbm, tab_hbm, out_hbm,
          idx_v, q0, q1, g0, g1, sel0, sel1,
          si, sg0, sg1, ss0, ss1)


def kernel(x, table):
    xt = x.T.astype(jnp.int32)
    tab = (table.T.reshape(_EMB, _TROWS, _PACK)
           .transpose(1, 2, 0).reshape(_TROWS, _PACK * _EMB))
    out_t = _embed_lookup(xt, tab)
    return out_t.transpose(2, 0, 1)


# speedup vs baseline: 1.5309x; 1.0993x over previous
import functools

import jax
import jax.numpy as jnp
from jax import lax
from jax.experimental import pallas as pl
from jax.experimental.pallas import tpu as pltpu
from jax.experimental.pallas import tpu_sc as plsc

_VOCAB = 1000000
_EMB = 32
_BATCH = 16384
_HIST = 50
_N = _BATCH * _HIST
_PACK = 4
_TROWS = _VOCAB // _PACK

_NC = 2
_NS = 16
_NW = _NC * _NS
_BBLK = _BATCH // _NW
_CHUNK = 256
_NCHUNK = _HIST * (_BBLK // _CHUNK)
_GROUPS = _CHUNK // 16


def _compute_q(h, bb0, idx_v, q_ref):
    @pl.loop(0, _GROUPS)
    def _(k):
        q_ref[pl.ds(k * 16, 16)] = lax.shift_right_logical(
            idx_v[h, pl.ds(bb0 + k * 16, 16)], 2)


def _select_chunk(h, bb0, idx_v, g_ref, sel_ref):
    lanes = lax.iota(jnp.int32, 16)
    for k in range(_GROUPS):
        rows16 = lanes + k * 16
        idx16 = idx_v[h, pl.ds(bb0 + k * 16, 16)]
        foff16 = (idx16 & (_PACK - 1)) * _EMB

        @pl.loop(0, _EMB, unroll=8)
        def _col(j):
            ed16 = (lanes + j) & (_EMB - 1)
            vals = plsc.load_gather(g_ref, [rows16, foff16 + ed16])
            plsc.store_scatter(sel_ref, [ed16, rows16], vals)


def _body(xt_hbm, tab_hbm, out_hbm,
          idx_v, q0, q1, g0, g1, sel0, sel1,
          si, sg0, sg1, ss0, ss1):
    wid = lax.axis_index("s") * _NC + lax.axis_index("c")
    bbase = wid * _BBLK
    q_v = (q0, q1)
    g_v = (g0, g1)
    sel_v = (sel0, sel1)
    sem_g = (sg0, sg1)
    sem_s = (ss0, ss1)

    def hb(c):
        return lax.shift_right_logical(c, 1), (c & 1) * _CHUNK

    def gather(b):
        return pltpu.async_copy(tab_hbm.at[q_v[b]], g_v[b], sem_g[b])

    def store(c, b):
        h, bb0 = hb(c)
        return pltpu.async_copy(
            sel_v[b], out_hbm.at[h, :, pl.ds(bbase + bb0, _CHUNK)],
            sem_s[b])

    def chunk_steady(c, b):
        nb = 1 - b

        @pl.when(c + 1 < _NCHUNK)
        def _():
            h1, bb1 = hb(c + 1)
            _compute_q(h1, bb1, idx_v, q_v[nb])
            gather(nb)

        pltpu.make_async_copy(tab_hbm.at[q_v[b]], g_v[b], sem_g[b]).wait()

        @pl.when(c >= 2)
        def _():
            pltpu.make_async_copy(
                sel_v[b], out_hbm.at[0, :, pl.ds(bbase, _CHUNK)],
                sem_s[b]).wait()

        h, bb0 = hb(c)
        _select_chunk(h, bb0, idx_v, g_v[b], sel_v[b])
        store(c, b)

    pltpu.async_copy(xt_hbm.at[:, pl.ds(bbase, _BBLK)], idx_v, si).wait()
    _compute_q(0, 0, idx_v, q_v[0])
    gather(0)

    @pl.loop(0, _NCHUNK, step=2)
    def _(c):
        chunk_steady(c, 0)
        chunk_steady(c + 1, 1)

    pltpu.make_async_copy(
        sel_v[0], out_hbm.at[0, :, pl.ds(bbase, _CHUNK)], sem_s[0]).wait()
    pltpu.make_async_copy(
        sel_v[1], out_hbm.at[0, :, pl.ds(bbase, _CHUNK)], sem_s[1]).wait()


@functools.partial(
    pl.kernel,
    mesh=plsc.VectorSubcoreMesh(core_axis_name="c", subcore_axis_name="s"),
    compiler_params=pltpu.CompilerParams(needs_layout_passes=False),
    out_type=jax.ShapeDtypeStruct((_HIST, _EMB, _BATCH), jnp.float32),
    scratch_types=[
        pltpu.VMEM((_HIST, _BBLK), jnp.int32),
        pltpu.VMEM((_CHUNK,), jnp.int32),
        pltpu.VMEM((_CHUNK,), jnp.int32),
        pltpu.VMEM((_CHUNK, 128), jnp.float32),
        pltpu.VMEM((_CHUNK, 128), jnp.float32),
        pltpu.VMEM((_EMB, _CHUNK), jnp.float32),
        pltpu.VMEM((_EMB, _CHUNK), jnp.float32),
        pltpu.SemaphoreType.DMA,
        pltpu.SemaphoreType.DMA,
        pltpu.SemaphoreType.DMA,
        pltpu.SemaphoreType.DMA,
        pltpu.SemaphoreType.DMA,
    ],
)
def _embed_lookup(xt_hbm, tab_hbm, out_hbm,
                  idx_v, q0, q1, g0, g1, sel0, sel1,
                  si, sg0, sg1, ss0, ss1):
    _body(xt_hbm, tab_hbm, out_hbm,
          idx_v, q0, q1, g0, g1, sel0, sel1,
          si, sg0, sg1, ss0, ss1)


_VBLK = 128
_NBLK = _VOCAB // _VBLK
_BLK_PER_W = _NBLK // _NW
_TAIL_V = _NBLK * _VBLK
_TAIL_N = _VOCAB - _TAIL_V


def _pack_block(k, tin, tout):
    lanes = lax.iota(jnp.int32, 16)
    l4 = lax.shift_right_logical(lanes, 2)
    c1 = l4 * 4 + (lanes & 3)
    ls32 = (lanes & 3) * _EMB
    for r0 in range(0, _EMB, 4):
        vv16 = c1 + 4 * r0
        r16 = l4 + r0

        @pl.loop(0, _EMB, unroll=8)
        def _(e0):
            e16 = (lanes + e0) & (_EMB - 1)
            vals = plsc.load_gather(tin, [e16, vv16])
            plsc.store_scatter(tout, [r16, ls32 + e16], vals)


def _pack_body(tabt_hbm, tail_hbm, tab2_hbm, tin0, tin1, tout0, tout1,
               sr0, sr1, sw0, sw1):
    wid = lax.axis_index("s") * _NC + lax.axis_index("c")
    tin = (tin0, tin1)
    tout = (tout0, tout1)
    sem_r = (sr0, sr1)
    sem_w = (sw0, sw1)

    def read(k, b):
        return pltpu.async_copy(
            tabt_hbm.at[:, pl.ds(k * _VBLK, _VBLK)], tin[b], sem_r[b])

    def write(k, b):
        return pltpu.async_copy(
            tout[b], tab2_hbm.at[pl.ds(k * _EMB, _EMB)], sem_w[b])

    nb = _BLK_PER_W + jnp.where(wid < 4, 1, 0)
    k0 = wid * _BLK_PER_W
    read(k0, 0).wait()

    def do_block(t, b, k):
        @pl.when(t + 1 < nb)
        def _():
            kn = jnp.where(t + 1 < _BLK_PER_W, k0 + t + 1,
                           _NW * _BLK_PER_W + wid)
            read(kn, 1 - b)

        @pl.when(t >= 2)
        def _():
            pltpu.make_async_copy(
                tout[b], tab2_hbm.at[pl.ds(0, _EMB)], sem_w[b]).wait()

        _pack_block(k, tin[b], tout[b])
        write(k, b)

        @pl.when(t + 1 < nb)
        def _():
            pltpu.make_async_copy(
                tabt_hbm.at[:, pl.ds(0, _VBLK)], tin[1 - b], sem_r[1 - b]
            ).wait()

    @pl.loop(0, _BLK_PER_W, step=2)
    def _(t):
        do_block(t, 0, k0 + t)
        do_block(t + 1, 1, k0 + t + 1)

    @pl.when(wid < 4)
    def _():
        do_block(_BLK_PER_W, 0, _NW * _BLK_PER_W + wid)

    @pl.when(wid == _NW - 1)
    def _():
        def scoped(tl_v, sem):
            pltpu.async_copy(tail_hbm, tl_v, sem).wait()
            lanes = lax.iota(jnp.int32, 16)
            l4 = lax.shift_right_logical(lanes, 2)
            c1 = l4 * 4 + (lanes & 3)
            ls32 = (lanes & 3) * _EMB
            for r0 in range(0, _TAIL_N // _PACK, 4):
                vv16 = c1 + 4 * r0
                r16 = l4 + r0

                @pl.loop(0, _EMB)
                def _(e0):
                    e16 = (lanes + e0) & (_EMB - 1)
                    vals = plsc.load_gather(tl_v, [vv16, e16])
                    plsc.store_scatter(tout0, [r16, ls32 + e16], vals)
            pltpu.async_copy(
                tout0.at[pl.ds(0, _TAIL_N // _PACK)],
                tab2_hbm.at[pl.ds(_TAIL_V // _PACK, _TAIL_N // _PACK)],
                sem).wait()

        pl.run_scoped(scoped, pltpu.VMEM((_TAIL_N, _EMB), jnp.float32),
                      pltpu.SemaphoreType.DMA)

    pltpu.make_async_copy(
        tout[0], tab2_hbm.at[pl.ds(0, _EMB)], sem_w[0]).wait()
    pltpu.make_async_copy(
        tout[1], tab2_hbm.at[pl.ds(0, _EMB)], sem_w[1]).wait()


@functools.partial(
    pl.kernel,
    mesh=plsc.VectorSubcoreMesh(core_axis_name="c", subcore_axis_name="s"),
    compiler_params=pltpu.CompilerParams(needs_layout_passes=False),
    out_type=jax.ShapeDtypeStruct((_TROWS, _PACK * _EMB), jnp.float32),
    scratch_types=[
        pltpu.VMEM((_EMB, _VBLK), jnp.float32),
        pltpu.VMEM((_EMB, _VBLK), jnp.float32),
        pltpu.VMEM((_EMB, 128), jnp.float32),
        pltpu.VMEM((_EMB, 128), jnp.float32),
        pltpu.SemaphoreType.DMA,
        pltpu.SemaphoreType.DMA,
        pltpu.SemaphoreType.DMA,
        pltpu.SemaphoreType.DMA,
    ],
)
def _pack_table(tabt_hbm, tail_hbm, tab2_hbm, tin0, tin1, tout0, tout1,
                sr0, sr1, sw0, sw1):
    _pack_body(tabt_hbm, tail_hbm, tab2_hbm, tin0, tin1, tout0, tout1,
               sr0, sr1, sw0, sw1)


def kernel(x, table):
    xt = x.T.astype(jnp.int32)
    tab = _pack_table(table.T, table[_TAIL_V:, :])
    out_t = _embed_lookup(xt, tab)
    return out_t.transpose(2, 0, 1)


# speedup vs baseline: 2.6188x; 1.7106x over previous
import functools

import jax
import jax.numpy as jnp
from jax import lax
from jax.experimental import pallas as pl
from jax.experimental.pallas import tpu as pltpu
from jax.experimental.pallas import tpu_sc as plsc

_VOCAB = 1000000
_EMB = 32
_BATCH = 16384
_HIST = 50
_N = _BATCH * _HIST
_PACK = 4
_TROWS = _VOCAB // _PACK

_NC = 2
_NS = 16
_NW = _NC * _NS
_BBLK = _BATCH // _NW
_CHUNK = 256
_NCHUNK = _HIST * (_BBLK // _CHUNK)
_GROUPS = _CHUNK // 16


def _compute_q(h, bb0, idx_v, q_ref):
    @pl.loop(0, _GROUPS)
    def _(k):
        q_ref[pl.ds(k * 16, 16)] = lax.shift_right_logical(
            idx_v[h, pl.ds(bb0 + k * 16, 16)], 2)


def _select_chunk(h, bb0, idx_v, g_ref, sel_ref):
    lanes = lax.iota(jnp.int32, 16)
    for k in range(_GROUPS):
        rows16 = lanes + k * 16
        idx16 = idx_v[h, pl.ds(bb0 + k * 16, 16)]
        foff16 = (idx16 & (_PACK - 1)) * _EMB

        @functools.partial(plsc.parallel_loop, 0, _EMB, unroll=8)
        def _col(j):
            ed16 = (lanes + j) & (_EMB - 1)
            vals = plsc.load_gather(g_ref, [rows16, foff16 + ed16])
            plsc.store_scatter(sel_ref, [ed16, rows16], vals)


def _body(xt_hbm, tab_hbm, out_hbm,
          idx_v, q0, q1, g0, g1, sel0, sel1,
          si, sg0, sg1, ss0, ss1):
    wid = lax.axis_index("s") * _NC + lax.axis_index("c")
    bbase = wid * _BBLK
    q_v = (q0, q1)
    g_v = (g0, g1)
    sel_v = (sel0, sel1)
    sem_g = (sg0, sg1)
    sem_s = (ss0, ss1)

    def hb(c):
        return lax.shift_right_logical(c, 1), (c & 1) * _CHUNK

    def gather(b):
        return pltpu.async_copy(tab_hbm.at[q_v[b]], g_v[b], sem_g[b])

    def store(c, b):
        h, bb0 = hb(c)
        return pltpu.async_copy(
            sel_v[b], out_hbm.at[h, :, pl.ds(bbase + bb0, _CHUNK)],
            sem_s[b])

    def chunk_steady(c, b):
        nb = 1 - b

        @pl.when(c + 1 < _NCHUNK)
        def _():
            h1, bb1 = hb(c + 1)
            _compute_q(h1, bb1, idx_v, q_v[nb])
            gather(nb)

        pltpu.make_async_copy(tab_hbm.at[q_v[b]], g_v[b], sem_g[b]).wait()

        @pl.when(c >= 2)
        def _():
            pltpu.make_async_copy(
                sel_v[b], out_hbm.at[0, :, pl.ds(bbase, _CHUNK)],
                sem_s[b]).wait()

        h, bb0 = hb(c)
        _select_chunk(h, bb0, idx_v, g_v[b], sel_v[b])
        store(c, b)

    pltpu.async_copy(xt_hbm.at[:, pl.ds(bbase, _BBLK)], idx_v, si).wait()
    _compute_q(0, 0, idx_v, q_v[0])
    gather(0)

    @pl.loop(0, _NCHUNK, step=2)
    def _(c):
        chunk_steady(c, 0)
        chunk_steady(c + 1, 1)

    pltpu.make_async_copy(
        sel_v[0], out_hbm.at[0, :, pl.ds(bbase, _CHUNK)], sem_s[0]).wait()
    pltpu.make_async_copy(
        sel_v[1], out_hbm.at[0, :, pl.ds(bbase, _CHUNK)], sem_s[1]).wait()


@functools.partial(
    pl.kernel,
    mesh=plsc.VectorSubcoreMesh(core_axis_name="c", subcore_axis_name="s"),
    compiler_params=pltpu.CompilerParams(needs_layout_passes=False),
    out_type=jax.ShapeDtypeStruct((_HIST, _EMB, _BATCH), jnp.float32),
    scratch_types=[
        pltpu.VMEM((_HIST, _BBLK), jnp.int32),
        pltpu.VMEM((_CHUNK,), jnp.int32),
        pltpu.VMEM((_CHUNK,), jnp.int32),
        pltpu.VMEM((_CHUNK, 128), jnp.float32),
        pltpu.VMEM((_CHUNK, 128), jnp.float32),
        pltpu.VMEM((_EMB, _CHUNK), jnp.float32),
        pltpu.VMEM((_EMB, _CHUNK), jnp.float32),
        pltpu.SemaphoreType.DMA,
        pltpu.SemaphoreType.DMA,
        pltpu.SemaphoreType.DMA,
        pltpu.SemaphoreType.DMA,
        pltpu.SemaphoreType.DMA,
    ],
)
def _embed_lookup(xt_hbm, tab_hbm, out_hbm,
                  idx_v, q0, q1, g0, g1, sel0, sel1,
                  si, sg0, sg1, ss0, ss1):
    _body(xt_hbm, tab_hbm, out_hbm,
          idx_v, q0, q1, g0, g1, sel0, sel1,
          si, sg0, sg1, ss0, ss1)


_VBLK = 128
_NBLK = _VOCAB // _VBLK
_BLK_PER_W = _NBLK // _NW
_TAIL_V = _NBLK * _VBLK
_TAIL_N = _VOCAB - _TAIL_V


def _pack_block(k, tin, tout):
    lanes = lax.iota(jnp.int32, 16)
    l4 = lax.shift_right_logical(lanes, 2)
    c1 = l4 * 4 + (lanes & 3)
    ls32 = (lanes & 3) * _EMB
    for r0 in range(0, _EMB, 4):
        vv16 = c1 + 4 * r0
        r16 = l4 + r0

        @functools.partial(plsc.parallel_loop, 0, _EMB, unroll=8)
        def _(e0):
            e16 = (lanes + e0) & (_EMB - 1)
            vals = plsc.load_gather(tin, [e16, vv16])
            plsc.store_scatter(tout, [r16, ls32 + e16], vals)


def _pack_body(tabt_hbm, tail_hbm, tab2_hbm, tin0, tin1, tout0, tout1,
               sr0, sr1, sw0, sw1):
    wid = lax.axis_index("s") * _NC + lax.axis_index("c")
    tin = (tin0, tin1)
    tout = (tout0, tout1)
    sem_r = (sr0, sr1)
    sem_w = (sw0, sw1)

    def read(k, b):
        return pltpu.async_copy(
            tabt_hbm.at[:, pl.ds(k * _VBLK, _VBLK)], tin[b], sem_r[b])

    def write(k, b):
        return pltpu.async_copy(
            tout[b], tab2_hbm.at[pl.ds(k * _EMB, _EMB)], sem_w[b])

    nb = _BLK_PER_W + jnp.where(wid < 4, 1, 0)
    k0 = wid * _BLK_PER_W
    read(k0, 0).wait()

    def do_block(t, b, k):
        @pl.when(t + 1 < nb)
        def _():
            kn = jnp.where(t + 1 < _BLK_PER_W, k0 + t + 1,
                           _NW * _BLK_PER_W + wid)
            read(kn, 1 - b)

        @pl.when(t >= 2)
        def _():
            pltpu.make_async_copy(
                tout[b], tab2_hbm.at[pl.ds(0, _EMB)], sem_w[b]).wait()

        _pack_block(k, tin[b], tout[b])
        write(k, b)

        @pl.when(t + 1 < nb)
        def _():
            pltpu.make_async_copy(
                tabt_hbm.at[:, pl.ds(0, _VBLK)], tin[1 - b], sem_r[1 - b]
            ).wait()

    @pl.loop(0, _BLK_PER_W, step=2)
    def _(t):
        do_block(t, 0, k0 + t)
        do_block(t + 1, 1, k0 + t + 1)

    @pl.when(wid < 4)
    def _():
        do_block(_BLK_PER_W, 0, _NW * _BLK_PER_W + wid)

    @pl.when(wid == _NW - 1)
    def _():
        def scoped(tl_v, sem):
            pltpu.async_copy(tail_hbm, tl_v, sem).wait()
            lanes = lax.iota(jnp.int32, 16)
            l4 = lax.shift_right_logical(lanes, 2)
            c1 = l4 * 4 + (lanes & 3)
            ls32 = (lanes & 3) * _EMB
            for r0 in range(0, _TAIL_N // _PACK, 4):
                vv16 = c1 + 4 * r0
                r16 = l4 + r0

                @pl.loop(0, _EMB)
                def _(e0):
                    e16 = (lanes + e0) & (_EMB - 1)
                    vals = plsc.load_gather(tl_v, [vv16, e16])
                    plsc.store_scatter(tout0, [r16, ls32 + e16], vals)
            pltpu.async_copy(
                tout0.at[pl.ds(0, _TAIL_N // _PACK)],
                tab2_hbm.at[pl.ds(_TAIL_V // _PACK, _TAIL_N // _PACK)],
                sem).wait()

        pl.run_scoped(scoped, pltpu.VMEM((_TAIL_N, _EMB), jnp.float32),
                      pltpu.SemaphoreType.DMA)

    pltpu.make_async_copy(
        tout[0], tab2_hbm.at[pl.ds(0, _EMB)], sem_w[0]).wait()
    pltpu.make_async_copy(
        tout[1], tab2_hbm.at[pl.ds(0, _EMB)], sem_w[1]).wait()


@functools.partial(
    pl.kernel,
    mesh=plsc.VectorSubcoreMesh(core_axis_name="c", subcore_axis_name="s"),
    compiler_params=pltpu.CompilerParams(needs_layout_passes=False),
    out_type=jax.ShapeDtypeStruct((_TROWS, _PACK * _EMB), jnp.float32),
    scratch_types=[
        pltpu.VMEM((_EMB, _VBLK), jnp.float32),
        pltpu.VMEM((_EMB, _VBLK), jnp.float32),
        pltpu.VMEM((_EMB, 128), jnp.float32),
        pltpu.VMEM((_EMB, 128), jnp.float32),
        pltpu.SemaphoreType.DMA,
        pltpu.SemaphoreType.DMA,
        pltpu.SemaphoreType.DMA,
        pltpu.SemaphoreType.DMA,
    ],
)
def _pack_table(tabt_hbm, tail_hbm, tab2_hbm, tin0, tin1, tout0, tout1,
                sr0, sr1, sw0, sw1):
    _pack_body(tabt_hbm, tail_hbm, tab2_hbm, tin0, tin1, tout0, tout1,
               sr0, sr1, sw0, sw1)


def kernel(x, table):
    xt = x.T.astype(jnp.int32)
    tab = _pack_table(table.T, table[_TAIL_V:, :])
    out_t = _embed_lookup(xt, tab)
    return out_t.transpose(2, 0, 1)
